# baseline (speedup 1.0000x reference)
import functools

import jax
import jax.numpy as jnp
from jax import lax
from jax.experimental import pallas as pl
from jax.experimental.pallas import tpu as pltpu
from jax.experimental.pallas import tpu_sc as plsc
from jax.sharding import Mesh, PartitionSpec as P

_N = 1024
_D = 128
_NC = 2
_NS = 16
_NW = _NC * _NS
_HS = 512
_HD = _N - _HS


@functools.lru_cache(maxsize=None)
def _build(R):
    rp = R // _NW
    g_rows = R + 1024
    stage = g_rows // _NS
    win = _HS + rp

    def body(tbl_hbm, out_hbm, idx_v, stage_v, win_v, shared, sem, sem2):
        c = lax.axis_index("c")
        s = lax.axis_index("s")
        w = s * _NC + c

        k0 = s * stage
        for t in range(stage // 16):
            v = (R + 1022 - k0 - 16 * t) - lax.iota(jnp.int32, 16)
            idx_v[pl.ds(16 * t, 16)] = jnp.maximum(v, 0)
        pltpu.async_copy(tbl_hbm.at[idx_v], stage_v, sem).wait()
        pltpu.sync_copy(stage_v, shared.at[pl.ds(k0, stage)])
        plsc.subcore_barrier()

        w0 = R - rp * (w + 1)
        pltpu.sync_copy(shared.at[pl.ds(w0, win)], win_v)

        def row(j, carry):
            i = w * rp + j
            cp = pltpu.async_copy(
                win_v.at[pl.ds(rp - 1 - j, _HS)],
                out_hbm.at[i, pl.ds(0, _HS)], sem2)
            pltpu.sync_copy(shared.at[pl.ds(R - 1 - i + _HS, _HD)],
                            out_hbm.at[i, pl.ds(_HS, _HD)])
            cp.wait()
            return carry

        lax.fori_loop(0, rp, row, 0)

    return pl.kernel(
        body,
        out_type=jax.ShapeDtypeStruct((R, _N, _D), jnp.float32),
        mesh=plsc.VectorSubcoreMesh(
            core_axis_name="c", subcore_axis_name="s",
            num_cores=_NC, num_subcores=_NS),
        scratch_types=[
            pltpu.VMEM((stage,), jnp.int32),
            pltpu.VMEM((stage, _D), jnp.float32),
            pltpu.VMEM((win, _D), jnp.float32),
            pltpu.VMEM_SHARED((g_rows, _D), jnp.float32),
            pltpu.SemaphoreType.DMA,
            pltpu.SemaphoreType.DMA,
        ],
    )


def kernel(seq_len, rel_pos_emb):
    del seq_len
    devs = jax.devices()
    ndev = 2 if len(devs) >= 2 else 1
    R = _N // ndev
    rpe = _build(R)
    if ndev == 1:
        return rpe(rel_pos_emb)

    def shard_fn(tbl):
        d = lax.axis_index("x")
        td = lax.dynamic_slice(tbl, (R * d, 0), (R + 1023, _D))
        return rpe(td)

    return jax.shard_map(
        shard_fn,
        mesh=Mesh(devs[:ndev], ("x",)),
        in_specs=P(None, None),
        out_specs=P("x", None, None),
    )(rel_pos_emb)


# speedup vs baseline: 1.2738x; 1.2738x over previous
import functools

import jax
import jax.numpy as jnp
from jax import lax
from jax.experimental import pallas as pl
from jax.experimental.pallas import tpu as pltpu
from jax.experimental.pallas import tpu_sc as plsc

_N = 1024
_D = 128
_NC = 2
_NS = 16
_NW = _NC * _NS
_ROWS_PER_W = _N // _NW
_STAGE = 2048 // _NS
_HS = 512
_HD = _N - _HS
_WIN = _HS + _ROWS_PER_W


def _rpe_body(tbl_hbm, out_hbm, idx_v, idxw_v, stage_v, win_v, shared,
              sem, sem2):
    c = lax.axis_index("c")
    s = lax.axis_index("s")
    w = s * _NC + c
    w0 = (_N - _ROWS_PER_W) - _ROWS_PER_W * w

    k0 = s * _STAGE
    for t in range(_STAGE // 16):
        v = (2046 - k0 - 16 * t) - lax.iota(jnp.int32, 16)
        idx_v[pl.ds(16 * t, 16)] = jnp.maximum(v, 0)
    for t in range(_WIN // 16):
        idxw_v[pl.ds(16 * t, 16)] = (2046 - w0 - 16 * t) - lax.iota(
            jnp.int32, 16)
    cp_a = pltpu.async_copy(tbl_hbm.at[idx_v], stage_v, sem)
    win_cps = [
        pltpu.async_copy(tbl_hbm.at[idxw_v.at[pl.ds(128 * q, 128)]],
                         win_v.at[pl.ds(128 * q, 128)], sem2)
        for q in range(_WIN // 128)
    ]
    win_cps.append(
        pltpu.async_copy(tbl_hbm.at[idxw_v.at[pl.ds(512, _WIN - 512)]],
                         win_v.at[pl.ds(512, _WIN - 512)], sem2))
    cp_a.wait()
    pltpu.sync_copy(stage_v, shared.at[pl.ds(k0, _STAGE)])
    plsc.subcore_barrier()
    for cp in win_cps:
        cp.wait()

    def body(j, carry):
        i = w * _ROWS_PER_W + j
        cp = pltpu.async_copy(
            win_v.at[pl.ds(_ROWS_PER_W - 1 - j, _HS)],
            out_hbm.at[i, pl.ds(0, _HS)], sem2)
        pltpu.sync_copy(shared.at[pl.ds(1023 - i + _HS, _HD)],
                        out_hbm.at[i, pl.ds(_HS, _HD)])
        cp.wait()
        return carry

    lax.fori_loop(0, _ROWS_PER_W, body, 0)


_rpe = functools.partial(
    pl.kernel,
    out_type=jax.ShapeDtypeStruct((_N, _N, _D), jnp.float32),
    mesh=plsc.VectorSubcoreMesh(
        core_axis_name="c", subcore_axis_name="s",
        num_cores=_NC, num_subcores=_NS),
    scratch_types=[
        pltpu.VMEM((_STAGE,), jnp.int32),
        pltpu.VMEM((_WIN,), jnp.int32),
        pltpu.VMEM((_STAGE, _D), jnp.float32),
        pltpu.VMEM((_WIN, _D), jnp.float32),
        pltpu.VMEM_SHARED((2048, _D), jnp.float32),
        pltpu.SemaphoreType.DMA,
        pltpu.SemaphoreType.DMA,
    ],
)(_rpe_body)


def kernel(seq_len, rel_pos_emb):
    del seq_len
    return _rpe(rel_pos_emb)


# speedup vs baseline: 1.2833x; 1.0074x over previous
import functools

import jax
import jax.numpy as jnp
from jax import lax
from jax.experimental import pallas as pl
from jax.experimental.pallas import tpu as pltpu
from jax.experimental.pallas import tpu_sc as plsc

_N = 1024
_D = 128
_NC = 2
_NS = 16
_NW = _NC * _NS
_ROWS_PER_W = _N // _NW
_STAGE = 2048 // _NS
_HS = 512
_HD = _N - _HS
_WIN = _HS + _ROWS_PER_W


def _rpe_body(tbl_hbm, out_hbm, idx_v, stage_v, win_v, shared, sem, sem2):
    c = lax.axis_index("c")
    s = lax.axis_index("s")
    w = s * _NC + c

    k0 = s * _STAGE
    for t in range(_STAGE // 16):
        v = (2046 - k0 - 16 * t) - lax.iota(jnp.int32, 16)
        idx_v[pl.ds(16 * t, 16)] = jnp.maximum(v, 0)
    pltpu.async_copy(tbl_hbm.at[idx_v], stage_v, sem).wait()
    pltpu.sync_copy(stage_v, shared.at[pl.ds(k0, _STAGE)])
    plsc.subcore_barrier()

    w0 = (_N - _ROWS_PER_W) - _ROWS_PER_W * w
    pltpu.sync_copy(shared.at[pl.ds(w0, _WIN)], win_v)

    def body(j, carry):
        i = w * _ROWS_PER_W + j
        cp = pltpu.async_copy(
            win_v.at[pl.ds(_ROWS_PER_W - 1 - j, _HS)],
            out_hbm.at[i, pl.ds(0, _HS)], sem2)
        pltpu.sync_copy(shared.at[pl.ds(1023 - i + _HS, _HD)],
                        out_hbm.at[i, pl.ds(_HS, _HD)])
        cp.wait()
        return carry

    lax.fori_loop(0, _ROWS_PER_W, body, 0)


_rpe = functools.partial(
    pl.kernel,
    out_type=jax.ShapeDtypeStruct((_N, _N, _D), jnp.float32),
    mesh=plsc.VectorSubcoreMesh(
        core_axis_name="c", subcore_axis_name="s",
        num_cores=_NC, num_subcores=_NS),
    scratch_types=[
        pltpu.VMEM((_STAGE,), jnp.int32),
        pltpu.VMEM((_STAGE, _D), jnp.float32),
        pltpu.VMEM((_WIN, _D), jnp.float32),
        pltpu.VMEM_SHARED((2048, _D), jnp.float32),
        pltpu.SemaphoreType.DMA,
        pltpu.SemaphoreType.DMA,
    ],
)(_rpe_body)


def kernel(seq_len, rel_pos_emb):
    del seq_len
    return _rpe(rel_pos_emb)
